# Initial kernel scaffold; baseline (speedup 1.0000x reference)
import jax
import jax.numpy as jnp
from jax.experimental import pallas as pl


def kernel(x, diao, diaoemb_weight, phemb_weight):
    raise NotImplementedError("write your pallas kernel here")


# speedup vs baseline: 1.3489x; 1.3489x over previous
import functools

import jax
import jax.numpy as jnp
from jax import lax
from jax.experimental import pallas as pl
from jax.experimental.pallas import tpu as pltpu
from jax.experimental.pallas import tpu_sc as plsc

N_EMBD = 1024
VOCAB = 5
NPAIR = VOCAB * VOCAB

ROWS = 4 * 8192
NW = 32
RPW = ROWS // NW
CB = 32
NCH = RPW // CB
LANES = 16


def _combine_body(d_ref, p_ref, out_ref):
    d = d_ref[...]
    p = p_ref[...]
    out_ref[...] = (d[:, None, :] + p[None, :, :]).reshape(NPAIR, N_EMBD)


def _combine(diaoemb_weight, phemb_weight):
    return pl.pallas_call(
        _combine_body,
        out_shape=jax.ShapeDtypeStruct((NPAIR, N_EMBD), jnp.float32),
    )(diaoemb_weight, phemb_weight)


def _sc_body(comb_hbm, x_hbm, diao_hbm, out_hbm, xv, dv, idxv, bufs,
             gsem, wsem0, wsem1):
    wid = lax.axis_index("s") * 2 + lax.axis_index("c")
    base = wid * RPW

    pltpu.sync_copy(x_hbm.at[pl.ds(base, RPW)], xv)
    pltpu.sync_copy(diao_hbm.at[pl.ds(base, RPW)], dv)

    for k in range(RPW // LANES):
        s = pl.ds(k * LANES, LANES)
        idxv[s] = dv[s] * VOCAB + xv[s]

    wsems = (wsem0, wsem1)
    writes = [None, None]
    for c in range(NCH):
        b = c % 2
        if writes[b] is not None:
            writes[b].wait()
        pltpu.async_copy(
            comb_hbm.at[idxv.at[pl.ds(c * CB, CB)]], bufs.at[b], gsem
        ).wait()
        writes[b] = pltpu.async_copy(
            bufs.at[b], out_hbm.at[pl.ds(base + c * CB, CB)], wsems[b]
        )
    writes[0].wait()
    writes[1].wait()


_sc_lookup = functools.partial(
    pl.kernel,
    out_type=jax.ShapeDtypeStruct((ROWS, N_EMBD), jnp.float32),
    mesh=plsc.VectorSubcoreMesh(core_axis_name="c", subcore_axis_name="s"),
    scratch_types=[
        pltpu.VMEM((RPW,), jnp.int32),
        pltpu.VMEM((RPW,), jnp.int32),
        pltpu.VMEM((RPW,), jnp.int32),
        pltpu.VMEM((2, CB, N_EMBD), jnp.float32),
        pltpu.SemaphoreType.DMA,
        pltpu.SemaphoreType.DMA,
        pltpu.SemaphoreType.DMA,
    ],
)(_sc_body)


@jax.jit
def kernel(x, diao, diaoemb_weight, phemb_weight):
    comb = _combine(diaoemb_weight, phemb_weight)
    xf = x.reshape(ROWS).astype(jnp.int32)
    df = diao.reshape(ROWS).astype(jnp.int32)
    out = _sc_lookup(comb, xf, df)
    return out.reshape(x.shape[0], x.shape[1], N_EMBD)


# speedup vs baseline: 1.3504x; 1.0012x over previous
import functools

import jax
import jax.numpy as jnp
from jax import lax
from jax.experimental import pallas as pl
from jax.experimental.pallas import tpu as pltpu
from jax.experimental.pallas import tpu_sc as plsc

N_EMBD = 1024
VOCAB = 5
NPAIR = VOCAB * VOCAB
NPAD = 32

ROWS = 4 * 8192
NW = 32
RPW = ROWS // NW
CB = 32
NCH = RPW // CB
LANES = 16


def _combine_body(d_ref, p_ref, out_ref):
    d = d_ref[...]
    p = p_ref[...]
    comb = (d[:, None, :] + p[None, :, :]).reshape(NPAIR, N_EMBD)
    pad = jnp.zeros((NPAD - NPAIR, N_EMBD), jnp.float32)
    out_ref[...] = jnp.concatenate([comb, pad], axis=0)


def _combine(diaoemb_weight, phemb_weight):
    return pl.pallas_call(
        _combine_body,
        out_shape=jax.ShapeDtypeStruct((NPAD, N_EMBD), jnp.float32),
    )(diaoemb_weight, phemb_weight)


def _sc_body(comb_hbm, x_hbm, diao_hbm, out_hbm, xv, dv, idxv, bufs,
             gsem0, gsem1, wsem0, wsem1):
    sid = lax.axis_index("s")
    wid = sid * 2 + lax.axis_index("c")
    base = wid * RPW

    pltpu.sync_copy(x_hbm.at[pl.ds(base, RPW)], xv)
    pltpu.sync_copy(diao_hbm.at[pl.ds(base, RPW)], dv)

    for k in range(RPW // LANES):
        s = pl.ds(k * LANES, LANES)
        idxv[s] = dv[s] * VOCAB + xv[s]

    gsems = (gsem0, gsem1)
    wsems = (wsem0, wsem1)
    gets = [None, None]
    writes = [None, None]

    def fire_gather(c):
        b = c % 2
        gets[b] = pltpu.async_copy(
            comb_hbm.at[idxv.at[pl.ds(c * CB, CB)]], bufs.at[b], gsems[b]
        )

    fire_gather(0)
    for c in range(NCH):
        b = c % 2
        if c + 1 < NCH:
            if writes[1 - b] is not None:
                writes[1 - b].wait()
            fire_gather(c + 1)
        gets[b].wait()
        writes[b] = pltpu.async_copy(
            bufs.at[b], out_hbm.at[pl.ds(base + c * CB, CB)], wsems[b]
        )
    writes[0].wait()
    writes[1].wait()


_sc_lookup = functools.partial(
    pl.kernel,
    out_type=jax.ShapeDtypeStruct((ROWS, N_EMBD), jnp.float32),
    mesh=plsc.VectorSubcoreMesh(core_axis_name="c", subcore_axis_name="s"),
    scratch_types=[
        pltpu.VMEM((RPW,), jnp.int32),
        pltpu.VMEM((RPW,), jnp.int32),
        pltpu.VMEM((RPW,), jnp.int32),
        pltpu.VMEM((2, CB, N_EMBD), jnp.float32),
        pltpu.SemaphoreType.DMA,
        pltpu.SemaphoreType.DMA,
        pltpu.SemaphoreType.DMA,
        pltpu.SemaphoreType.DMA,
    ],
)(_sc_body)


@jax.jit
def kernel(x, diao, diaoemb_weight, phemb_weight):
    comb = _combine(diaoemb_weight, phemb_weight)
    xf = x.reshape(ROWS).astype(jnp.int32)
    df = diao.reshape(ROWS).astype(jnp.int32)
    out = _sc_lookup(comb, xf, df)
    return out.reshape(x.shape[0], x.shape[1], N_EMBD)


# speedup vs baseline: 1.5977x; 1.1831x over previous
import functools

import jax
import jax.numpy as jnp
from jax import lax
from jax.experimental import pallas as pl
from jax.experimental.pallas import tpu as pltpu
from jax.experimental.pallas import tpu_sc as plsc

N_EMBD = 1024
VOCAB = 5
NPAIR = VOCAB * VOCAB
NPAD = 32

ROWS = 4 * 8192
NW = 32
RPW = ROWS // NW
CB = 32
NCH = RPW // CB
LANES = 16


def _combine_body(d_ref, p_ref, out_ref):
    d = d_ref[...]
    p = p_ref[...]
    comb = (d[:, None, :] + p[None, :, :]).reshape(NPAIR, N_EMBD)
    pad = jnp.zeros((NPAD - NPAIR, N_EMBD), jnp.float32)
    out_ref[...] = jnp.concatenate([comb, pad], axis=0)


def _combine(diaoemb_weight, phemb_weight):
    return pl.pallas_call(
        _combine_body,
        out_shape=jax.ShapeDtypeStruct((NPAD, N_EMBD), jnp.float32),
    )(diaoemb_weight, phemb_weight)


def _sc_body(comb_hbm, x_hbm, diao_hbm, out_hbm, xv, dv, idxv, bufs, comb_v,
             wsem0, wsem1):
    sid = lax.axis_index("s")
    wid = sid * 2 + lax.axis_index("c")
    base = wid * RPW

    pltpu.sync_copy(comb_hbm, comb_v)

    pltpu.sync_copy(x_hbm.at[pl.ds(base, RPW)], xv)
    pltpu.sync_copy(diao_hbm.at[pl.ds(base, RPW)], dv)

    for k in range(RPW // LANES):
        s = pl.ds(k * LANES, LANES)
        idxv[s] = dv[s] * VOCAB + xv[s]

    wsems = (wsem0, wsem1)
    writes = [None, None]
    for c in range(NCH):
        b = c % 2
        if writes[b] is not None:
            writes[b].wait()

        def fill_row(r, _):
            iv = idxv[pl.ds(c * CB + r, LANES)]
            row = iv[0]
            for k in range(N_EMBD // LANES):
                s = pl.ds(k * LANES, LANES)
                bufs[b, r, s] = comb_v[row, s]
            return 0

        lax.fori_loop(0, CB, fill_row, 0, unroll=False)
        writes[b] = pltpu.async_copy(
            bufs.at[b], out_hbm.at[pl.ds(base + c * CB, CB)], wsems[b]
        )
    writes[0].wait()
    writes[1].wait()


_sc_lookup = functools.partial(
    pl.kernel,
    out_type=jax.ShapeDtypeStruct((ROWS, N_EMBD), jnp.float32),
    mesh=plsc.VectorSubcoreMesh(core_axis_name="c", subcore_axis_name="s"),
    scratch_types=[
        pltpu.VMEM((RPW,), jnp.int32),
        pltpu.VMEM((RPW,), jnp.int32),
        pltpu.VMEM((RPW + LANES,), jnp.int32),
        pltpu.VMEM((2, CB, N_EMBD), jnp.float32),
        pltpu.VMEM((NPAD, N_EMBD), jnp.float32),
        pltpu.SemaphoreType.DMA,
        pltpu.SemaphoreType.DMA,
    ],
)(_sc_body)


@jax.jit
def kernel(x, diao, diaoemb_weight, phemb_weight):
    comb = _combine(diaoemb_weight, phemb_weight)
    xf = x.reshape(ROWS).astype(jnp.int32)
    df = diao.reshape(ROWS).astype(jnp.int32)
    out = _sc_lookup(comb, xf, df)
    return out.reshape(x.shape[0], x.shape[1], N_EMBD)
